# baseline (speedup 1.0000x reference)
import functools

import jax
import jax.numpy as jnp
import numpy as np
from jax import lax
from jax.experimental import pallas as pl
from jax.experimental.pallas import tpu as pltpu
from jax.experimental.pallas import tpu_sc as plsc

_N = 10000
_E = 320000
_D = 128
_H = 8
_NB = 4
_NA = 2
_G = 128
_NW = 32
_NG = 16
_FPW = 4
_EHALF = _E // 2
_CH = 2000
_NCHUNK = _EHALF // _CH
_STEPS = _CH // 16

_Pnp = np.zeros((8, 64, 128), np.float32)
_Qnp = np.zeros((8, 128, 128), np.float32)
for _m in range(8):
    for _h in range(8):
        _Pnp[_m, _h * 8 + _m, _h * 16:(_h + 1) * 16] = 1.0
        _Qnp[_m, 16 * _m:16 * _m + 16, _h * 16:(_h + 1) * 16] = np.eye(16, dtype=np.float32)


def _pre_body(x_ref, bw_ref, cw_ref, cb_ref, b_out, w_out):
    x = x_ref[...]
    b_out[...] = jnp.dot(x, bw_ref[...], preferred_element_type=jnp.float32)
    w_out[...] = jnp.dot(x, cw_ref[...], preferred_element_type=jnp.float32) + cb_ref[...]


def _pre(x, bases_w, comb_w, comb_b):
    return pl.pallas_call(
        _pre_body,
        out_shape=(
            jax.ShapeDtypeStruct((_N, _NB * 16), jnp.float32),
            jax.ShapeDtypeStruct((_N, _H * _NB * _NA), jnp.float32),
        ),
    )(x, bases_w, comb_w, comb_b.reshape(1, -1))


def _take(x, idx):
    return x.at[idx].get(mode="promise_in_bounds")


def _sc_agg(src, dst, bases_t):
    mesh = plsc.VectorSubcoreMesh(core_axis_name="c", subcore_axis_name="s")

    @functools.partial(
        pl.kernel,
        mesh=mesh,
        compiler_params=pltpu.CompilerParams(needs_layout_passes=False),
        out_type=(
            jax.ShapeDtypeStruct((_NW, _FPW * _N), jnp.float32),
            jax.ShapeDtypeStruct((_NW, _FPW * _N), jnp.float32),
        ),
        scratch_types=[
            pltpu.VMEM((_FPW * _N,), jnp.float32),
            pltpu.VMEM((_FPW * _N,), jnp.float32),
            pltpu.VMEM((_FPW * _N,), jnp.float32),
            pltpu.VMEM((_CH,), jnp.int32),
            pltpu.VMEM((_CH,), jnp.int32),
        ],
    )
    def body(src_h, dst_h, bt_h, osum_h, omax_h, bas_v, asum_v, amax_v, src_v, dst_v):
        wid = lax.axis_index("s") * 2 + lax.axis_index("c")
        grp = wid // 2
        half = wid % 2
        ebase = half * _EHALF
        pltpu.sync_copy(bt_h.at[grp], bas_v)

        def initf(j, carry):
            asum_v[pl.ds(j * 16, 16)] = jnp.zeros((16,), jnp.float32)
            amax_v[pl.ds(j * 16, 16)] = jnp.full((16,), -jnp.inf, jnp.float32)
            return carry

        lax.fori_loop(0, _FPW * _N // 16, initf, 0)
        lane = lax.iota(jnp.int32, 16)

        def chunkf(c, carry):
            pltpu.sync_copy(src_h.at[pl.ds(ebase + c * _CH, _CH)], src_v)
            pltpu.sync_copy(dst_h.at[pl.ds(ebase + c * _CH, _CH)], dst_v)

            def stepf(i, carry2):
                d = dst_v[pl.ds(i * 16, 16)]
                s = src_v[pl.ds(i * 16, 16)]
                dk, sv = plsc.sort_key_val(d, s)
                same = []
                for sh in (1, 2, 4, 8):
                    idxs = jnp.maximum(lane - sh, 0)
                    same.append((_take(dk, idxs) == dk) & (lane >= sh))
                is_end = (_take(dk, jnp.minimum(lane + 1, 15)) != dk) | (lane == 15)
                for f in range(_FPW):
                    val = plsc.load_gather(bas_v, [sv * _FPW + f])
                    addr = dk * _FPW + f
                    plsc.addupdate_scatter(asum_v, [addr], val)
                    vmax = val
                    for t, sh in enumerate((1, 2, 4, 8)):
                        idxs = jnp.maximum(lane - sh, 0)
                        vmax = jnp.where(same[t], jnp.maximum(vmax, _take(vmax, idxs)), vmax)
                    cur = plsc.load_gather(amax_v, [addr])
                    plsc.store_scatter(amax_v, [addr], jnp.maximum(cur, vmax), mask=is_end)
                return carry2

            lax.fori_loop(0, _STEPS, stepf, 0)
            return carry

        lax.fori_loop(0, _NCHUNK, chunkf, 0)
        pltpu.sync_copy(asum_v, osum_h.at[wid])
        pltpu.sync_copy(amax_v, omax_h.at[wid])

    return body(src, dst, bases_t)


def _post_body(suma_ref, sumb_ref, maxa_ref, maxb_ref, bas_ref, wgt_ref,
               batch_ref, p_ref, q_ref, cb_ref, gw_ref, gb_ref, gms_ref,
               out_ref):
    bases = bas_ref[...]
    agg_sum = suma_ref[...] + sumb_ref[...] + bases
    agg_max = jnp.maximum(jnp.maximum(maxa_ref[...], maxb_ref[...]), bases)
    cat = jnp.concatenate([agg_sum, agg_max], axis=1)
    wgt = wgt_ref[...]
    h = jnp.broadcast_to(cb_ref[...], (_N, _D))
    for m in range(8):
        h = h + jnp.dot(wgt, p_ref[m], preferred_element_type=jnp.float32) * \
            jnp.dot(cat, q_ref[m], preferred_element_type=jnp.float32)

    oh = (batch_ref[...] == lax.broadcasted_iota(jnp.int32, (_N, _G), 1)
          ).astype(jnp.float32)
    dn = (((0,), (0,)), ((), ()))
    cnt = lax.dot_general(oh, jnp.ones((_N, 1), jnp.float32), dn,
                          preferred_element_type=jnp.float32)
    cnt = jnp.maximum(cnt, 1.0)
    mean = lax.dot_general(oh, h, dn, preferred_element_type=jnp.float32) / cnt
    out = h - jnp.dot(oh, mean, preferred_element_type=jnp.float32) * gms_ref[...]
    var = lax.dot_general(oh, out * out, dn, preferred_element_type=jnp.float32) / cnt
    std = jnp.sqrt(var + 1e-5)
    stdb = jnp.dot(oh, std, preferred_element_type=jnp.float32)
    out_ref[...] = jnp.maximum(gw_ref[...] * out / stdb + gb_ref[...], 0.0)


def _post(sum_a, sum_b, max_a, max_b, bases, weightings, batch2d, conv_b,
          gn_w, gn_b, gn_ms):
    return pl.pallas_call(
        _post_body,
        out_shape=jax.ShapeDtypeStruct((_N, _D), jnp.float32),
    )(sum_a, sum_b, max_a, max_b, bases, weightings, batch2d,
      jnp.asarray(_Pnp), jnp.asarray(_Qnp),
      conv_b.reshape(1, -1), gn_w.reshape(1, -1), gn_b.reshape(1, -1),
      gn_ms.reshape(1, -1))


def kernel(node, edge_index, edge_attr, batch_ptr, bases_w, comb_w, comb_b,
           conv_b, gn_w, gn_b, gn_ms):
    del edge_attr
    bases, weightings = _pre(node, bases_w, comb_w, comb_b)
    bases_t = bases.reshape(_N, _NG, _FPW).transpose(1, 0, 2).reshape(_NG, _FPW * _N)
    s_sum, s_max = _sc_agg(edge_index[0], edge_index[1], bases_t)

    def _split(p):
        p = p.reshape(_NG, 2, _N, _FPW)
        a = p[:, 0].transpose(1, 0, 2).reshape(_N, _NG * _FPW)
        b = p[:, 1].transpose(1, 0, 2).reshape(_N, _NG * _FPW)
        return a, b

    sum_a, sum_b = _split(s_sum)
    max_a, max_b = _split(s_max)
    return _post(sum_a, sum_b, max_a, max_b, bases, weightings,
                 batch_ptr.reshape(_N, 1).astype(jnp.int32),
                 conv_b, gn_w, gn_b, gn_ms)


# speedup vs baseline: 1.7283x; 1.7283x over previous
import functools

import jax
import jax.numpy as jnp
import numpy as np
from jax import lax
from jax.experimental import pallas as pl
from jax.experimental.pallas import tpu as pltpu
from jax.experimental.pallas import tpu_sc as plsc

_N = 10000
_E = 320000
_D = 128
_H = 8
_NB = 4
_NA = 2
_G = 128
_NW = 32
_FPW = 2
_CH = 20000
_NCHUNK = _E // _CH
_STEPS = _CH // 16

_Pnp = np.zeros((8, 64, 128), np.float32)
_Qnp = np.zeros((8, 128, 128), np.float32)
for _m in range(8):
    for _h in range(8):
        _Pnp[_m, _h * 8 + _m, _h * 16:(_h + 1) * 16] = 1.0
        _Qnp[_m, 16 * _m:16 * _m + 16, _h * 16:(_h + 1) * 16] = np.eye(16, dtype=np.float32)


def _pre_body(x_ref, bw_ref, cw_ref, cb_ref, b_out, w_out):
    x = x_ref[...]
    b_out[...] = jnp.dot(x, bw_ref[...], preferred_element_type=jnp.float32)
    w_out[...] = jnp.dot(x, cw_ref[...], preferred_element_type=jnp.float32) + cb_ref[...]


def _pre(x, bases_w, comb_w, comb_b):
    return pl.pallas_call(
        _pre_body,
        out_shape=(
            jax.ShapeDtypeStruct((_N, _NB * 16), jnp.float32),
            jax.ShapeDtypeStruct((_N, _H * _NB * _NA), jnp.float32),
        ),
    )(x, bases_w, comb_w, comb_b.reshape(1, -1))


def _take(x, idx):
    return x.at[idx].get(mode="promise_in_bounds")


def _sc_agg(src, dst, bases_t):
    mesh = plsc.VectorSubcoreMesh(core_axis_name="c", subcore_axis_name="s")

    @functools.partial(
        pl.kernel,
        mesh=mesh,
        compiler_params=pltpu.CompilerParams(needs_layout_passes=False),
        out_type=(
            jax.ShapeDtypeStruct((_NW * _FPW, _N), jnp.float32),
            jax.ShapeDtypeStruct((_NW * _FPW, _N), jnp.float32),
        ),
        scratch_types=[
            pltpu.VMEM((_N,), jnp.float32),
            pltpu.VMEM((_N,), jnp.float32),
            pltpu.VMEM((_N,), jnp.float32),
            pltpu.VMEM((_N,), jnp.float32),
            pltpu.VMEM((_N,), jnp.float32),
            pltpu.VMEM((_N,), jnp.float32),
            pltpu.VMEM((_CH,), jnp.int32),
            pltpu.VMEM((_CH,), jnp.int32),
        ],
    )
    def body(src_h, dst_h, bt_h, osum_h, omax_h,
             bas0_v, bas1_v, asum0_v, asum1_v, amax0_v, amax1_v, src_v, dst_v):
        wid = lax.axis_index("s") * 2 + lax.axis_index("c")
        pltpu.sync_copy(bt_h.at[wid * 2], bas0_v)
        pltpu.sync_copy(bt_h.at[wid * 2 + 1], bas1_v)
        fsets = ((bas0_v, asum0_v, amax0_v), (bas1_v, asum1_v, amax1_v))

        def initf(j, carry):
            for _, asm, amx in fsets:
                asm[pl.ds(j * 16, 16)] = jnp.zeros((16,), jnp.float32)
                amx[pl.ds(j * 16, 16)] = jnp.full((16,), -jnp.inf, jnp.float32)
            return carry

        lax.fori_loop(0, _N // 16, initf, 0)
        lane = lax.iota(jnp.int32, 16)

        def chunkf(c, carry):
            pltpu.sync_copy(src_h.at[pl.ds(c * _CH, _CH)], src_v)
            pltpu.sync_copy(dst_h.at[pl.ds(c * _CH, _CH)], dst_v)

            def stepf(i, carry2):
                d = dst_v[pl.ds(i * 16, 16)]
                s = src_v[pl.ds(i * 16, 16)]
                dk, sv = plsc.sort_key_val(d, s)
                same = []
                for sh in (1, 2, 4, 8):
                    idxs = jnp.maximum(lane - sh, 0)
                    same.append((_take(dk, idxs) == dk) & (lane >= sh))
                is_end = (_take(dk, jnp.minimum(lane + 1, 15)) != dk) | (lane == 15)
                for bas, asm, amx in fsets:
                    val = plsc.load_gather(bas, [sv])
                    plsc.addupdate_scatter(asm, [dk], val)
                    cur = plsc.load_gather(amx, [dk])
                    vmax = val
                    for t, sh in enumerate((1, 2, 4, 8)):
                        idxs = jnp.maximum(lane - sh, 0)
                        vmax = jnp.where(same[t], jnp.maximum(vmax, _take(vmax, idxs)), vmax)
                    plsc.store_scatter(amx, [dk], jnp.maximum(cur, vmax), mask=is_end)
                return carry2

            lax.fori_loop(0, _STEPS, stepf, 0)
            return carry

        lax.fori_loop(0, _NCHUNK, chunkf, 0)
        pltpu.sync_copy(asum0_v, osum_h.at[wid * 2])
        pltpu.sync_copy(asum1_v, osum_h.at[wid * 2 + 1])
        pltpu.sync_copy(amax0_v, omax_h.at[wid * 2])
        pltpu.sync_copy(amax1_v, omax_h.at[wid * 2 + 1])

    return body(src, dst, bases_t)


def _post_body(sum_ref, max_ref, bas_ref, wgt_ref, batch_ref, p_ref, q_ref,
               cb_ref, gw_ref, gb_ref, gms_ref, out_ref):
    bases = bas_ref[...]
    agg_sum = sum_ref[...] + bases
    agg_max = jnp.maximum(max_ref[...], bases)
    cat = jnp.concatenate([agg_sum, agg_max], axis=1)
    wgt = wgt_ref[...]
    h = jnp.broadcast_to(cb_ref[...], (_N, _D))
    for m in range(8):
        h = h + jnp.dot(wgt, p_ref[m], preferred_element_type=jnp.float32) * \
            jnp.dot(cat, q_ref[m], preferred_element_type=jnp.float32)

    oh = (batch_ref[...] == lax.broadcasted_iota(jnp.int32, (_N, _G), 1)
          ).astype(jnp.float32)
    dn = (((0,), (0,)), ((), ()))
    cnt = lax.dot_general(oh, jnp.ones((_N, 1), jnp.float32), dn,
                          preferred_element_type=jnp.float32)
    cnt = jnp.maximum(cnt, 1.0)
    mean = lax.dot_general(oh, h, dn, preferred_element_type=jnp.float32) / cnt
    out = h - jnp.dot(oh, mean, preferred_element_type=jnp.float32) * gms_ref[...]
    var = lax.dot_general(oh, out * out, dn, preferred_element_type=jnp.float32) / cnt
    std = jnp.sqrt(var + 1e-5)
    stdb = jnp.dot(oh, std, preferred_element_type=jnp.float32)
    out_ref[...] = jnp.maximum(gw_ref[...] * out / stdb + gb_ref[...], 0.0)


def _post(agg_sum, agg_max, bases, weightings, batch2d, conv_b, gn_w, gn_b, gn_ms):
    return pl.pallas_call(
        _post_body,
        out_shape=jax.ShapeDtypeStruct((_N, _D), jnp.float32),
    )(agg_sum, agg_max, bases, weightings, batch2d,
      jnp.asarray(_Pnp), jnp.asarray(_Qnp),
      conv_b.reshape(1, -1), gn_w.reshape(1, -1), gn_b.reshape(1, -1),
      gn_ms.reshape(1, -1))


def kernel(node, edge_index, edge_attr, batch_ptr, bases_w, comb_w, comb_b,
           conv_b, gn_w, gn_b, gn_ms):
    del edge_attr
    bases, weightings = _pre(node, bases_w, comb_w, comb_b)
    s_sum, s_max = _sc_agg(edge_index[0], edge_index[1], bases.T)
    return _post(s_sum.T, s_max.T, bases, weightings,
                 batch_ptr.reshape(_N, 1).astype(jnp.int32),
                 conv_b, gn_w, gn_b, gn_ms)
